# baseline (speedup 1.0000x reference)
import functools

import jax
import jax.numpy as jnp
from jax import lax
from jax.experimental import pallas as pl
from jax.experimental.pallas import tpu as pltpu
from jax.experimental.pallas import tpu_sc as plsc

_N = 10000
_D = 128
_NPAD = 10240
_ROWS = _NPAD // 128
_CH = 80
_NCHUNK = 128
_NCK0 = 256
_NCK1 = 0
_EPT = _CH * _NCHUNK
_EPAD = 32 * _EPT
_ECH = _EPAD // _CH
_DCH = 128
_DNCHUNK = 80
_SL = _NPAD // 16
_NSL = _NPAD // 16

_mesh = plsc.VectorSubcoreMesh(core_axis_name="c", subcore_axis_name="s")


def _deg_body(src_hbm, dst_hbm, ew_hbm, out_hbm, srcall, dstall, ewall, zbuf, degs, degd, sem):
    c = lax.axis_index("c")
    s = lax.axis_index("s")
    wid = c * 16 + s
    rowbase = wid * _DNCHUNK
    pltpu.sync_copy(src_hbm.at[pl.ds(rowbase, _DNCHUNK)], srcall)
    pltpu.sync_copy(dst_hbm.at[pl.ds(rowbase, _DNCHUNK)], dstall)
    pltpu.sync_copy(ew_hbm.at[pl.ds(rowbase, _DNCHUNK)], ewall)
    for i in range(_SL // 16):
        zbuf[pl.ds(i * 16, 16)] = jnp.zeros((16,), jnp.float32)
    pltpu.sync_copy(zbuf, degs.at[pl.ds(s * _SL, _SL)])
    pltpu.sync_copy(zbuf, degd.at[pl.ds(s * _SL, _SL)])
    plsc.subcore_barrier()

    def fire(i, carry):
        pltpu.async_copy(ewall.at[i], degs.at[srcall.at[i]], sem, add=True)
        pltpu.async_copy(ewall.at[i], degd.at[dstall.at[i]], sem, add=True)
        return carry

    lax.fori_loop(0, _DNCHUNK, fire, 0)

    def drain(i, carry):
        pltpu.make_async_copy(ewall.at[i], degs.at[srcall.at[i]], sem).wait()
        pltpu.make_async_copy(ewall.at[i], degd.at[dstall.at[i]], sem).wait()
        return carry

    lax.fori_loop(0, _DNCHUNK, drain, 0)
    plsc.subcore_barrier()
    pltpu.sync_copy(degs.at[pl.ds(s * _SL, _SL)], out_hbm.at[c, 0, pl.ds(s * _SL, _SL)])
    pltpu.sync_copy(degd.at[pl.ds(s * _SL, _SL)], out_hbm.at[c, 1, pl.ds(s * _SL, _SL)])


_deg_call = functools.partial(
    pl.kernel,
    mesh=_mesh,
    out_type=jax.ShapeDtypeStruct((2, 2, _NPAD), jnp.float32),
    scratch_types=[
        pltpu.VMEM((_DNCHUNK, _DCH), jnp.int32),
        pltpu.VMEM((_DNCHUNK, _DCH), jnp.int32),
        pltpu.VMEM((_DNCHUNK, _DCH), jnp.float32),
        pltpu.VMEM((_SL,), jnp.float32),
        pltpu.VMEM_SHARED((_NPAD,), jnp.float32),
        pltpu.VMEM_SHARED((_NPAD,), jnp.float32),
        pltpu.SemaphoreType.DMA,
    ],
)(_deg_body)


def _spmm_body(src_hbm, dst_hbm, ew_hbm, msg_hbm, out_hbm,
               s0, s1, s2, s3, s4, s5, s6, s7,
               d0, d1, d2, d3, d4, d5, d6, d7,
               w0, w1, w2, w3, w4, w5, w6, w7,
               r0, r1, r2, r3, acc,
               ie0, ie1, ie2, ie3, ie4, ie5, ie6, ie7,
               gs0, gs1, gs2, gs3, ss0, ss1, ss2, ss3):
    c = lax.axis_index("c")
    s = lax.axis_index("s")
    sbuf = (s0, s1, s2, s3, s4, s5, s6, s7)
    dbuf = (d0, d1, d2, d3, d4, d5, d6, d7)
    wbuf = (w0, w1, w2, w3, w4, w5, w6, w7)
    isem = (ie0, ie1, ie2, ie3, ie4, ie5, ie6, ie7)
    rbuf = (r0, r1, r2, r3)
    gsem = (gs0, gs1, gs2, gs3)
    ssem = (ss0, ss1, ss2, ss3)
    ebase = s * (_NCK0 * _CH)

    def zrow(r, carry):
        for f in range(_D // 16):
            r0[r, pl.ds(f * 16, 16)] = jnp.zeros((16,), jnp.float32)
        return carry

    lax.fori_loop(0, _CH, zrow, 0)
    for k in range(_SL // _CH):
        pltpu.sync_copy(r0, acc.at[pl.ds(s * _SL + k * _CH, _CH)])
    plsc.subcore_barrier()

    def start_idx(i, r):
        off = ebase + i * _CH
        pltpu.async_copy(src_hbm.at[pl.ds(off, _CH)], sbuf[r], isem[r])
        pltpu.async_copy(dst_hbm.at[pl.ds(off, _CH)], dbuf[r], isem[r])
        pltpu.async_copy(ew_hbm.at[pl.ds(off, _CH)], wbuf[r], isem[r])

    def wait_idx(i, r):
        off = ebase + i * _CH
        pltpu.make_async_copy(src_hbm.at[pl.ds(off, _CH)], sbuf[r], isem[r]).wait()
        pltpu.make_async_copy(dst_hbm.at[pl.ds(off, _CH)], dbuf[r], isem[r]).wait()
        pltpu.make_async_copy(ew_hbm.at[pl.ds(off, _CH)], wbuf[r], isem[r]).wait()

    def start_gather(q, r):
        pltpu.async_copy(msg_hbm.at[sbuf[r]], rbuf[q], gsem[q])

    def wait_gather(q, r):
        pltpu.make_async_copy(msg_hbm.at[sbuf[r]], rbuf[q], gsem[q]).wait()

    def start_scatter(q, r):
        pltpu.async_copy(rbuf[q], acc.at[dbuf[r]], ssem[q], add=True)

    def wait_scatter(q, r):
        pltpu.make_async_copy(rbuf[q], acc.at[dbuf[r]], ssem[q]).wait()

    def scale(q, r):
        def grp(g, carry):
            ewg = wbuf[r][pl.ds(g * 16, 16)]
            for i16 in range(16):
                e = g * 16 + i16
                w = ewg[i16]
                for f in range(_D // 16):
                    rbuf[q][e, pl.ds(f * 16, 16)] = rbuf[q][e, pl.ds(f * 16, 16)] * w
            return carry

        lax.fori_loop(0, _CH // 16, grp, 0)

    def chunk(i, t, first, last):
        q = t % 4
        wait_gather(q, t)
        if not first:
            wait_scatter((t + 2) % 4, (t + 6) % 8)
        if not last:
            start_idx(i + 5, (t + 5) % 8)
            wait_idx(i + 2, (t + 2) % 8)
            start_gather((t + 2) % 4, (t + 2) % 8)
        scale(q, t)
        start_scatter(q, t)

    @pl.when(c == 0)
    def _pipeline():
        for r in range(5):
            start_idx(r, r)
        wait_idx(0, 0)
        start_gather(0, 0)
        wait_idx(1, 1)
        start_gather(1, 1)

        for i in range(8):
            wait_gather(i % 4, i)
            if i >= 2:
                wait_scatter((i - 2) % 4, (i - 2) % 8)
            start_idx(i + 5, (i + 5) % 8)
            wait_idx(i + 2, (i + 2) % 8)
            start_gather((i + 2) % 4, (i + 2) % 8)
            scale(i % 4, i)
            start_scatter(i % 4, i)

        def body(j, carry):
            i0 = j * 8
            for t in range(8):
                chunk(i0 + t, t, False, False)
            return carry

        lax.fori_loop(1, _NCK0 // 8 - 1, body, 0)

        for k in range(8):
            i = _NCK0 - 8 + k
            wait_gather(k % 4, k % 8)
            wait_scatter((k - 2) % 4, (k - 2) % 8)
            if k + 5 < 8:
                start_idx(i + 5, (k + 5) % 8)
            if k + 2 < 8:
                wait_idx(i + 2, (k + 2) % 8)
                start_gather((k + 2) % 4, (k + 2) % 8)
            scale(k % 4, k % 8)
            start_scatter(k % 4, k % 8)
        wait_scatter(6 % 4, 6 % 8)
        wait_scatter(7 % 4, 7 % 8)
    plsc.subcore_barrier()
    pltpu.sync_copy(acc.at[pl.ds(s * _SL, _SL)], out_hbm.at[c, pl.ds(s * _SL, _SL)])


_spmm_call = functools.partial(
    pl.kernel,
    mesh=_mesh,
    out_type=jax.ShapeDtypeStruct((2, _NPAD, _D), jnp.float32),
    scratch_types=(
        [pltpu.VMEM((_CH,), jnp.int32)] * 16
        + [pltpu.VMEM((_CH,), jnp.float32)] * 8
        + [pltpu.VMEM((_CH, _D), jnp.float32)] * 4
        + [pltpu.VMEM_SHARED((_NPAD, _D), jnp.float32)]
        + [pltpu.SemaphoreType.DMA] * 16
    ),
)(_spmm_body)


def _rsqrt_body(degp_ref, rsd_ref):
    rsd_ref[...] = lax.rsqrt(degp_ref[0] + degp_ref[1] + 1e-6)


def _rsqrt_call(degp4):
    return pl.pallas_call(
        _rsqrt_body,
        out_shape=jax.ShapeDtypeStruct((2, _ROWS, 128), jnp.float32),
    )(degp4)


def _inproj_body(x_ref, win_ref, bin_ref, w1_ref, b1_ref, rs_ref, ms_ref):
    h = jnp.dot(x_ref[...], win_ref[...], preferred_element_type=jnp.float32) + bin_ref[...]
    m = jnp.dot(h, w1_ref[...], preferred_element_type=jnp.float32) + b1_ref[...]
    ms_ref[...] = m * rs_ref[...]


def _inproj_call(x, W_in, b_in, W1, b1, rsd_s):
    return pl.pallas_call(
        _inproj_body,
        out_shape=jax.ShapeDtypeStruct((_N, _D), jnp.float32),
    )(x, W_in, b_in, W1, b1, rsd_s)


def _mid_body(a0_ref, a1_ref, rd_ref, g_ref, be_ref, w_ref, b_ref, rs_ref, out_ref):
    cv = (a0_ref[...] + a1_ref[...]) * rd_ref[...]
    m = jnp.mean(cv, axis=0)
    v = jnp.mean((cv - m) ** 2, axis=0)
    h = jnp.maximum((cv - m) * lax.rsqrt(v + 1e-5) * g_ref[...] + be_ref[...], 0.0)
    out_ref[...] = (jnp.dot(h, w_ref[...], preferred_element_type=jnp.float32) + b_ref[...]) * rs_ref[...]


def _mid_call(a0, a1, rd, g, be, w, b, rs):
    return pl.pallas_call(
        _mid_body,
        out_shape=jax.ShapeDtypeStruct((_N, _D), jnp.float32),
    )(a0, a1, rd, g, be, w, b, rs)


def _fin_body(a0_ref, a1_ref, rd_ref, out_ref):
    out_ref[...] = (a0_ref[...] + a1_ref[...]) * rd_ref[...]


def _fin_call(a0, a1, rd):
    return pl.pallas_call(
        _fin_body,
        out_shape=jax.ShapeDtypeStruct((_N, _D), jnp.float32),
    )(a0, a1, rd)


def kernel(x, edge_index, edge_attr, W_in, b_in, W1, b1, W2, b2, W_mu, b_mu, W_lv, b_lv, g1, be1, g2, be2):
    src = edge_index[0]
    dst = edge_index[1]
    pad = _EPAD - src.shape[0]
    srcp = jnp.pad(src, (0, pad))
    dstp = jnp.pad(dst, (0, pad))
    ewp = jnp.pad(edge_attr, (0, pad))
    degp = _deg_call(srcp.reshape(_EPAD // _DCH, _DCH), dstp.reshape(_EPAD // _DCH, _DCH),
                     ewp.reshape(_EPAD // _DCH, _DCH))
    rsd = _rsqrt_call(degp.reshape(2, 2, _ROWS, 128))
    rsd_s = rsd[0].reshape(_NPAD, 1)[:_N]
    rsd_d = rsd[1].reshape(_NPAD, 1)[:_N]

    def spmm(ms):
        return _spmm_call(srcp, dstp, ewp, ms)

    ms1 = _inproj_call(x, W_in, b_in, W1, b1, rsd_s)
    acc1 = spmm(ms1)
    ms2 = _mid_call(acc1[0, :_N], acc1[1, :_N], rsd_d, g1, be1, W2, b2, rsd_s)
    acc2 = spmm(ms2)
    Wc = jnp.concatenate([W_mu, W_lv], axis=1)
    bc = jnp.concatenate([b_mu, b_lv])
    ms3 = _mid_call(acc2[0, :_N], acc2[1, :_N], rsd_d, g2, be2, Wc, bc, rsd_s)
    acc3 = spmm(ms3)
    full = _fin_call(acc3[0, :_N], acc3[1, :_N], rsd_d)
    return (full[:, :64], full[:, 64:])


# speedup vs baseline: 1.3970x; 1.3970x over previous
import functools

import jax
import jax.numpy as jnp
from jax import lax
from jax.experimental import pallas as pl
from jax.experimental.pallas import tpu as pltpu
from jax.experimental.pallas import tpu_sc as plsc

_N = 10000
_D = 128
_NPAD = 10240
_ROWS = _NPAD // 128
_CH = 80
_NCHUNK = 128
_NCK0 = 240
_NCK1 = 16
_EPT = _CH * _NCHUNK
_EPAD = 32 * _EPT
_ECH = _EPAD // _CH
_DCH = 128
_DNCHUNK = 80
_SL = _NPAD // 16
_NSL = _NPAD // 16

_mesh = plsc.VectorSubcoreMesh(core_axis_name="c", subcore_axis_name="s")


def _deg_body(src_hbm, dst_hbm, ew_hbm, out_hbm, srcall, dstall, ewall, zbuf, degs, degd, sem):
    c = lax.axis_index("c")
    s = lax.axis_index("s")
    wid = c * 16 + s
    rowbase = wid * _DNCHUNK
    pltpu.sync_copy(src_hbm.at[pl.ds(rowbase, _DNCHUNK)], srcall)
    pltpu.sync_copy(dst_hbm.at[pl.ds(rowbase, _DNCHUNK)], dstall)
    pltpu.sync_copy(ew_hbm.at[pl.ds(rowbase, _DNCHUNK)], ewall)
    for i in range(_SL // 16):
        zbuf[pl.ds(i * 16, 16)] = jnp.zeros((16,), jnp.float32)
    pltpu.sync_copy(zbuf, degs.at[pl.ds(s * _SL, _SL)])
    pltpu.sync_copy(zbuf, degd.at[pl.ds(s * _SL, _SL)])
    plsc.subcore_barrier()

    def fire(i, carry):
        pltpu.async_copy(ewall.at[i], degs.at[srcall.at[i]], sem, add=True)
        pltpu.async_copy(ewall.at[i], degd.at[dstall.at[i]], sem, add=True)
        return carry

    lax.fori_loop(0, _DNCHUNK, fire, 0)

    def drain(i, carry):
        pltpu.make_async_copy(ewall.at[i], degs.at[srcall.at[i]], sem).wait()
        pltpu.make_async_copy(ewall.at[i], degd.at[dstall.at[i]], sem).wait()
        return carry

    lax.fori_loop(0, _DNCHUNK, drain, 0)
    plsc.subcore_barrier()
    pltpu.sync_copy(degs.at[pl.ds(s * _SL, _SL)], out_hbm.at[c, 0, pl.ds(s * _SL, _SL)])
    pltpu.sync_copy(degd.at[pl.ds(s * _SL, _SL)], out_hbm.at[c, 1, pl.ds(s * _SL, _SL)])


_deg_call = functools.partial(
    pl.kernel,
    mesh=_mesh,
    out_type=jax.ShapeDtypeStruct((2, 2, _NPAD), jnp.float32),
    scratch_types=[
        pltpu.VMEM((_DNCHUNK, _DCH), jnp.int32),
        pltpu.VMEM((_DNCHUNK, _DCH), jnp.int32),
        pltpu.VMEM((_DNCHUNK, _DCH), jnp.float32),
        pltpu.VMEM((_SL,), jnp.float32),
        pltpu.VMEM_SHARED((_NPAD,), jnp.float32),
        pltpu.VMEM_SHARED((_NPAD,), jnp.float32),
        pltpu.SemaphoreType.DMA,
    ],
)(_deg_body)


def _spmm_body(src_hbm, dst_hbm, ew_hbm, msg_hbm, out_hbm,
               s0, s1, s2, s3, s4, s5, s6, s7,
               d0, d1, d2, d3, d4, d5, d6, d7,
               w0, w1, w2, w3, w4, w5, w6, w7,
               r0, r1, r2, r3, acc,
               ie0, ie1, ie2, ie3, ie4, ie5, ie6, ie7,
               gs0, gs1, gs2, gs3, ss0, ss1, ss2, ss3):
    c = lax.axis_index("c")
    s = lax.axis_index("s")
    sbuf = (s0, s1, s2, s3, s4, s5, s6, s7)
    dbuf = (d0, d1, d2, d3, d4, d5, d6, d7)
    wbuf = (w0, w1, w2, w3, w4, w5, w6, w7)
    isem = (ie0, ie1, ie2, ie3, ie4, ie5, ie6, ie7)
    rbuf = (r0, r1, r2, r3)
    gsem = (gs0, gs1, gs2, gs3)
    ssem = (ss0, ss1, ss2, ss3)
    nck = jnp.where(c == 0, _NCK0, _NCK1)
    ebase = (jnp.where(c == 0, s * _NCK0, 16 * _NCK0 + s * _NCK1)) * _CH

    def zrow(r, carry):
        for f in range(_D // 16):
            r0[r, pl.ds(f * 16, 16)] = jnp.zeros((16,), jnp.float32)
        return carry

    lax.fori_loop(0, _CH, zrow, 0)
    for k in range(_SL // _CH):
        pltpu.sync_copy(r0, acc.at[pl.ds(s * _SL + k * _CH, _CH)])
    plsc.subcore_barrier()

    def start_idx(i, r):
        off = ebase + i * _CH
        pltpu.async_copy(src_hbm.at[pl.ds(off, _CH)], sbuf[r], isem[r])
        pltpu.async_copy(dst_hbm.at[pl.ds(off, _CH)], dbuf[r], isem[r])
        pltpu.async_copy(ew_hbm.at[pl.ds(off, _CH)], wbuf[r], isem[r])

    def wait_idx(i, r):
        off = ebase + i * _CH
        pltpu.make_async_copy(src_hbm.at[pl.ds(off, _CH)], sbuf[r], isem[r]).wait()
        pltpu.make_async_copy(dst_hbm.at[pl.ds(off, _CH)], dbuf[r], isem[r]).wait()
        pltpu.make_async_copy(ew_hbm.at[pl.ds(off, _CH)], wbuf[r], isem[r]).wait()

    def start_gather(q, r):
        pltpu.async_copy(msg_hbm.at[sbuf[r]], rbuf[q], gsem[q])

    def wait_gather(q, r):
        pltpu.make_async_copy(msg_hbm.at[sbuf[r]], rbuf[q], gsem[q]).wait()

    def start_scatter(q, r):
        pltpu.async_copy(rbuf[q], acc.at[dbuf[r]], ssem[q], add=True)

    def wait_scatter(q, r):
        pltpu.make_async_copy(rbuf[q], acc.at[dbuf[r]], ssem[q]).wait()

    def scale(q, r):
        def grp(g, carry):
            ewg = wbuf[r][pl.ds(g * 16, 16)]
            for i16 in range(16):
                e = g * 16 + i16
                w = ewg[i16]
                for f in range(_D // 16):
                    rbuf[q][e, pl.ds(f * 16, 16)] = rbuf[q][e, pl.ds(f * 16, 16)] * w
            return carry

        lax.fori_loop(0, _CH // 16, grp, 0)

    def chunk(i, t, first, last):
        q = t % 4
        wait_gather(q, t)
        if not first:
            wait_scatter((t + 2) % 4, (t + 6) % 8)
        if not last:
            start_idx(i + 5, (t + 5) % 8)
            wait_idx(i + 2, (t + 2) % 8)
            start_gather((t + 2) % 4, (t + 2) % 8)
        scale(q, t)
        start_scatter(q, t)

    for r in range(5):
        start_idx(r, r)
    wait_idx(0, 0)
    start_gather(0, 0)
    wait_idx(1, 1)
    start_gather(1, 1)

    for i in range(8):
        wait_gather(i % 4, i)
        if i >= 2:
            wait_scatter((i - 2) % 4, (i - 2) % 8)
        start_idx(i + 5, (i + 5) % 8)
        if i >= 2:
            wait_idx(i + 2, (i + 2) % 8)
            start_gather((i + 2) % 4, (i + 2) % 8)
        else:
            wait_idx(i + 2, (i + 2) % 8)
            start_gather((i + 2) % 4, (i + 2) % 8)
        scale(i % 4, i)
        start_scatter(i % 4, i)

    def body(j, carry):
        i0 = j * 8
        for t in range(8):
            chunk(i0 + t, t, False, False)
        return carry

    lax.fori_loop(1, nck // 8 - 1, body, 0)

    for k in range(8):
        i = nck - 8 + k
        wait_gather(k % 4, k % 8)
        wait_scatter((k - 2) % 4, (k - 2) % 8)
        if k + 5 < 8:
            start_idx(i + 5, (k + 5) % 8)
        if k + 2 < 8:
            wait_idx(i + 2, (k + 2) % 8)
            start_gather((k + 2) % 4, (k + 2) % 8)
        scale(k % 4, k % 8)
        start_scatter(k % 4, k % 8)
    wait_scatter(6 % 4, 6 % 8)
    wait_scatter(7 % 4, 7 % 8)
    plsc.subcore_barrier()
    pltpu.sync_copy(acc.at[pl.ds(s * _SL, _SL)], out_hbm.at[c, pl.ds(s * _SL, _SL)])


_spmm_call = functools.partial(
    pl.kernel,
    mesh=_mesh,
    out_type=jax.ShapeDtypeStruct((2, _NPAD, _D), jnp.float32),
    scratch_types=(
        [pltpu.VMEM((_CH,), jnp.int32)] * 16
        + [pltpu.VMEM((_CH,), jnp.float32)] * 8
        + [pltpu.VMEM((_CH, _D), jnp.float32)] * 4
        + [pltpu.VMEM_SHARED((_NPAD, _D), jnp.float32)]
        + [pltpu.SemaphoreType.DMA] * 16
    ),
)(_spmm_body)


def _rsqrt_body(degp_ref, rsd_ref):
    rsd_ref[...] = lax.rsqrt(degp_ref[0] + degp_ref[1] + 1e-6)


def _rsqrt_call(degp4):
    return pl.pallas_call(
        _rsqrt_body,
        out_shape=jax.ShapeDtypeStruct((2, _ROWS, 128), jnp.float32),
    )(degp4)


def _inproj_body(x_ref, win_ref, bin_ref, w1_ref, b1_ref, rs_ref, ms_ref):
    h = jnp.dot(x_ref[...], win_ref[...], preferred_element_type=jnp.float32) + bin_ref[...]
    m = jnp.dot(h, w1_ref[...], preferred_element_type=jnp.float32) + b1_ref[...]
    ms_ref[...] = m * rs_ref[...]


def _inproj_call(x, W_in, b_in, W1, b1, rsd_s):
    return pl.pallas_call(
        _inproj_body,
        out_shape=jax.ShapeDtypeStruct((_N, _D), jnp.float32),
    )(x, W_in, b_in, W1, b1, rsd_s)


def _mid_body(a0_ref, a1_ref, rd_ref, g_ref, be_ref, w_ref, b_ref, rs_ref, out_ref):
    cv = (a0_ref[...] + a1_ref[...]) * rd_ref[...]
    m = jnp.mean(cv, axis=0)
    v = jnp.mean((cv - m) ** 2, axis=0)
    h = jnp.maximum((cv - m) * lax.rsqrt(v + 1e-5) * g_ref[...] + be_ref[...], 0.0)
    out_ref[...] = (jnp.dot(h, w_ref[...], preferred_element_type=jnp.float32) + b_ref[...]) * rs_ref[...]


def _mid_call(a0, a1, rd, g, be, w, b, rs):
    return pl.pallas_call(
        _mid_body,
        out_shape=jax.ShapeDtypeStruct((_N, _D), jnp.float32),
    )(a0, a1, rd, g, be, w, b, rs)


def _fin_body(a0_ref, a1_ref, rd_ref, out_ref):
    out_ref[...] = (a0_ref[...] + a1_ref[...]) * rd_ref[...]


def _fin_call(a0, a1, rd):
    return pl.pallas_call(
        _fin_body,
        out_shape=jax.ShapeDtypeStruct((_N, _D), jnp.float32),
    )(a0, a1, rd)


def kernel(x, edge_index, edge_attr, W_in, b_in, W1, b1, W2, b2, W_mu, b_mu, W_lv, b_lv, g1, be1, g2, be2):
    src = edge_index[0]
    dst = edge_index[1]
    pad = _EPAD - src.shape[0]
    srcp = jnp.pad(src, (0, pad))
    dstp = jnp.pad(dst, (0, pad))
    ewp = jnp.pad(edge_attr, (0, pad))
    degp = _deg_call(srcp.reshape(_EPAD // _DCH, _DCH), dstp.reshape(_EPAD // _DCH, _DCH),
                     ewp.reshape(_EPAD // _DCH, _DCH))
    rsd = _rsqrt_call(degp.reshape(2, 2, _ROWS, 128))
    rsd_s = rsd[0].reshape(_NPAD, 1)[:_N]
    rsd_d = rsd[1].reshape(_NPAD, 1)[:_N]

    def spmm(ms):
        return _spmm_call(srcp, dstp, ewp, ms)

    ms1 = _inproj_call(x, W_in, b_in, W1, b1, rsd_s)
    acc1 = spmm(ms1)
    ms2 = _mid_call(acc1[0, :_N], acc1[1, :_N], rsd_d, g1, be1, W2, b2, rsd_s)
    acc2 = spmm(ms2)
    Wc = jnp.concatenate([W_mu, W_lv], axis=1)
    bc = jnp.concatenate([b_mu, b_lv])
    ms3 = _mid_call(acc2[0, :_N], acc2[1, :_N], rsd_d, g2, be2, Wc, bc, rsd_s)
    acc3 = spmm(ms3)
    full = _fin_call(acc3[0, :_N], acc3[1, :_N], rsd_d)
    return (full[:, :64], full[:, 64:])
